# baseline (speedup 1.0000x reference)
import jax
import jax.numpy as jnp
from jax.experimental import pallas as pl
from jax.experimental.pallas import tpu as pltpu

_BLOCK = 2000
_T = 8


def _hetero_kernel(t_ref, x_ref, ws_ref, o_ref):
    B = x_ref.shape[0]
    d = x_ref.shape[1]
    xf = x_ref[...]
    tcol = t_ref[0, 0][:, None]
    tm = jax.lax.rem(tcol, 4)
    parts = [jnp.where(tm == g, xf, jnp.float32(0)).astype(jnp.bfloat16)
             for g in range(4)]
    lane = jax.lax.broadcasted_iota(jnp.int32, (B, d), 1)
    parts.append(
        jnp.where(tcol == lane, jnp.float32(1), jnp.float32(0))
        .astype(jnp.bfloat16))
    xcat = jnp.concatenate(parts, axis=1)
    y2 = jax.lax.dot_general(
        xcat, ws_ref[...],
        dimension_numbers=(((1,), (0,)), ((), ())),
        preferred_element_type=jnp.float32,
    )
    o_ref[...] = jnp.where(tcol < 4, y2[:, :d], y2[:, d:])


def kernel(x, node_type_list, W, b):
    N, d_in = x.shape
    d_out = W.shape[1]
    nb = N // _BLOCK
    t3d = node_type_list.astype(jnp.int32).reshape(nb, 1, _BLOCK)
    wt = jnp.transpose(W, (0, 2, 1))
    grp = [jnp.concatenate([wt[g], wt[g + 4]], axis=1) for g in range(4)]
    bb = jnp.concatenate([b, b], axis=1)
    brows = jnp.concatenate(
        [bb, jnp.zeros((d_in - _T, 2 * d_out), b.dtype)], axis=0)
    ws = jnp.concatenate(grp + [brows], axis=0).astype(jnp.bfloat16)
    K = ws.shape[0]
    return pl.pallas_call(
        _hetero_kernel,
        grid=(nb,),
        in_specs=[
            pl.BlockSpec((1, 1, _BLOCK), lambda i: (i, 0, 0)),
            pl.BlockSpec((_BLOCK, d_in), lambda i: (i, 0)),
            pl.BlockSpec((K, 2 * d_out), lambda i: (0, 0)),
        ],
        out_specs=pl.BlockSpec((_BLOCK, d_out), lambda i: (i, 0)),
        compiler_params=pltpu.CompilerParams(
            dimension_semantics=("parallel",)),
        out_shape=jax.ShapeDtypeStruct((N, d_out), x.dtype),
    )(t3d, x, ws)


# speedup vs baseline: 1.4332x; 1.4332x over previous
import jax
import jax.numpy as jnp
from jax.experimental import pallas as pl
from jax.experimental.pallas import tpu as pltpu

_BLOCK = 10000
_T = 8


def _hetero_kernel(t_ref, x_ref, ws_ref, o_ref):
    B = x_ref.shape[0]
    d = x_ref.shape[1]
    xf = x_ref[...]
    tcol = t_ref[0, 0][:, None]
    tm = jax.lax.rem(tcol, 4)
    parts = [jnp.where(tm == g, xf, jnp.float32(0)).astype(jnp.bfloat16)
             for g in range(4)]
    lane = jax.lax.broadcasted_iota(jnp.int32, (B, d), 1)
    parts.append(
        jnp.where(tcol == lane, jnp.float32(1), jnp.float32(0))
        .astype(jnp.bfloat16))
    xcat = jnp.concatenate(parts, axis=1)
    y2 = jax.lax.dot_general(
        xcat, ws_ref[...],
        dimension_numbers=(((1,), (0,)), ((), ())),
        preferred_element_type=jnp.float32,
    )
    o_ref[...] = jnp.where(tcol < 4, y2[:, :d], y2[:, d:])


def kernel(x, node_type_list, W, b):
    N, d_in = x.shape
    d_out = W.shape[1]
    nb = N // _BLOCK
    t3d = node_type_list.astype(jnp.int32).reshape(nb, 1, _BLOCK)
    wt = jnp.transpose(W, (0, 2, 1))
    grp = [jnp.concatenate([wt[g], wt[g + 4]], axis=1) for g in range(4)]
    bb = jnp.concatenate([b, b], axis=1)
    brows = jnp.concatenate(
        [bb, jnp.zeros((d_in - _T, 2 * d_out), b.dtype)], axis=0)
    ws = jnp.concatenate(grp + [brows], axis=0).astype(jnp.bfloat16)
    K = ws.shape[0]
    return pl.pallas_call(
        _hetero_kernel,
        grid=(nb,),
        in_specs=[
            pl.BlockSpec((1, 1, _BLOCK), lambda i: (i, 0, 0)),
            pl.BlockSpec((_BLOCK, d_in), lambda i: (i, 0)),
            pl.BlockSpec((K, 2 * d_out), lambda i: (0, 0)),
        ],
        out_specs=pl.BlockSpec((_BLOCK, d_out), lambda i: (i, 0)),
        compiler_params=pltpu.CompilerParams(
            dimension_semantics=("parallel",)),
        out_shape=jax.ShapeDtypeStruct((N, d_out), x.dtype),
    )(t3d, x, ws)
